# baseline (speedup 1.0000x reference)
import functools

import jax
import jax.numpy as jnp
from jax import lax
from jax.experimental import pallas as pl
from jax.experimental.pallas import tpu as pltpu
from jax.experimental.pallas import tpu_sc as plsc

VOCAB = 100000
HIDDEN = 1024
NTOK = 16384

NC = 2
NS = 16
NW = NC * NS
BPW = NTOK // NW
CHUNK = 32
NCHUNK = BPW // CHUNK
NBUF = 3

_mesh = plsc.VectorSubcoreMesh(core_axis_name="c", subcore_axis_name="s")


@functools.partial(
    pl.kernel,
    out_type=jax.ShapeDtypeStruct((NTOK, HIDDEN), jnp.float32),
    mesh=_mesh,
    scratch_types=[
        pltpu.VMEM((NCHUNK, CHUNK), jnp.int32),
        pltpu.VMEM((NBUF, CHUNK, HIDDEN), jnp.float32),
        pltpu.SemaphoreType.DMA,
        pltpu.SemaphoreType.DMA,
        pltpu.SemaphoreType.DMA,
        pltpu.SemaphoreType.DMA,
        pltpu.SemaphoreType.DMA,
        pltpu.SemaphoreType.DMA,
    ],
)
def _emb_lookup(x_hbm, table_hbm, out_hbm, idx_v, bufs,
                gsem0, gsem1, gsem2, ssem0, ssem1, ssem2):
    wid = lax.axis_index("s") * NC + lax.axis_index("c")
    base = wid * BPW

    pltpu.sync_copy(x_hbm.at[wid], idx_v)

    gsems = (gsem0, gsem1, gsem2)
    ssems = (ssem0, ssem1, ssem2)

    def gather(g):
        return pltpu.async_copy(
            table_hbm.at[idx_v.at[g]], bufs.at[g % NBUF], gsems[g % NBUF])

    def scatter(g):
        return pltpu.async_copy(
            bufs.at[g % NBUF], out_hbm.at[pl.ds(base + g * CHUNK, CHUNK)],
            ssems[g % NBUF])

    copies_g = [None] * NCHUNK
    copies_s = [None] * NCHUNK
    for g in range(NBUF):
        copies_g[g] = gather(g)
    for g in range(NCHUNK):
        copies_g[g].wait()
        copies_s[g] = scatter(g)
        j = g - (NBUF - 1)
        if j >= 0 and j + NBUF < NCHUNK:
            copies_s[j].wait()
            copies_g[j + NBUF] = gather(j + NBUF)
    for g in range(NCHUNK - NBUF, NCHUNK):
        copies_s[g].wait()


def kernel(x, table):
    ids = x.reshape(NW, NCHUNK, CHUNK).astype(jnp.int32)
    out = _emb_lookup(ids, table)
    return out.reshape(x.shape[0], x.shape[1], HIDDEN)


# speedup vs baseline: 1.0320x; 1.0320x over previous
import functools

import jax
import jax.numpy as jnp
from jax import lax
from jax.experimental import pallas as pl
from jax.experimental.pallas import tpu as pltpu
from jax.experimental.pallas import tpu_sc as plsc

VOCAB = 100000
HIDDEN = 1024
NTOK = 16384

NC = 2
NS = 16
NW = NC * NS
BPW = NTOK // NW
CHUNK = 32
NCHUNK = BPW // CHUNK

_mesh = plsc.VectorSubcoreMesh(core_axis_name="c", subcore_axis_name="s")


@functools.partial(
    pl.kernel,
    out_type=jax.ShapeDtypeStruct((NTOK, HIDDEN), jnp.float32),
    mesh=_mesh,
    scratch_types=[
        pltpu.VMEM((NCHUNK, CHUNK), jnp.int32),
        pltpu.VMEM((3, CHUNK, HIDDEN), jnp.float32),
        pltpu.SemaphoreType.DMA,
        pltpu.SemaphoreType.DMA,
        pltpu.SemaphoreType.DMA,
        pltpu.SemaphoreType.DMA,
        pltpu.SemaphoreType.DMA,
        pltpu.SemaphoreType.DMA,
    ],
)
def _emb_lookup(x_hbm, table_hbm, out_hbm, idx_v, bufs,
                gsem0, gsem1, gsem2, ssem0, ssem1, ssem2):
    NBUF = 3
    wid = lax.axis_index("s") * NC + lax.axis_index("c")
    base = wid * BPW

    pltpu.sync_copy(x_hbm.at[wid], idx_v)

    gsems = (gsem0, gsem1, gsem2)
    ssems = (ssem0, ssem1, ssem2)

    def gather(g):
        return pltpu.async_copy(
            table_hbm.at[idx_v.at[g]], bufs.at[g % NBUF], gsems[g % NBUF])

    def scatter(g):
        return pltpu.async_copy(
            bufs.at[g % NBUF], out_hbm.at[pl.ds(base + g * CHUNK, CHUNK)],
            ssems[g % NBUF])

    copies_g = [None] * NCHUNK
    copies_s = [None] * NCHUNK
    for g in range(NBUF):
        copies_g[g] = gather(g)
    for g in range(NCHUNK):
        copies_g[g].wait()
        copies_s[g] = scatter(g)
        if g + NBUF < NCHUNK:
            copies_s[g].wait()
            copies_g[g + NBUF] = gather(g + NBUF)
    for g in range(NCHUNK - NBUF, NCHUNK):
        copies_s[g].wait()


def kernel(x, table):
    ids = x.reshape(NW, NCHUNK, CHUNK).astype(jnp.int32)
    out = _emb_lookup(ids, table)
    return out.reshape(x.shape[0], x.shape[1], HIDDEN)


# speedup vs baseline: 1.0334x; 1.0014x over previous
import functools

import jax
import jax.numpy as jnp
from jax import lax
from jax.experimental import pallas as pl
from jax.experimental.pallas import tpu as pltpu
from jax.experimental.pallas import tpu_sc as plsc

VOCAB = 100000
HIDDEN = 1024
NTOK = 16384

NC = 2
NS = 16
NW = NC * NS
BPW = NTOK // NW
CHUNK = 32
NCHUNK = BPW // CHUNK

_mesh = plsc.VectorSubcoreMesh(core_axis_name="c", subcore_axis_name="s")


@functools.partial(
    pl.kernel,
    out_type=jax.ShapeDtypeStruct((NTOK, HIDDEN), jnp.float32),
    mesh=_mesh,
    scratch_types=[
        pltpu.VMEM((NCHUNK, CHUNK), jnp.int32),
        pltpu.VMEM((3, CHUNK, HIDDEN), jnp.float32),
        pltpu.SemaphoreType.DMA((3,)),
        pltpu.SemaphoreType.DMA((3,)),
    ],
)
def _emb_lookup(x_hbm, table_hbm, out_hbm, idx_v, bufs, gsem, ssem):
    NBUF = 3
    wid = lax.axis_index("s") * NC + lax.axis_index("c")
    base = wid * BPW

    pltpu.sync_copy(x_hbm.at[wid], idx_v)

    def gather(g):
        return pltpu.async_copy(
            table_hbm.at[idx_v.at[g]], bufs.at[g % NBUF], gsem.at[g % NBUF])

    def scatter(g):
        return pltpu.async_copy(
            bufs.at[g % NBUF], out_hbm.at[pl.ds(base + g * CHUNK, CHUNK)],
            ssem.at[g % NBUF])

    copies_g = [None] * NCHUNK
    copies_s = [None] * NCHUNK
    for g in range(NBUF):
        copies_g[g] = gather(g)
    for g in range(NCHUNK):
        copies_g[g].wait()
        copies_s[g] = scatter(g)
        if g + NBUF < NCHUNK:
            copies_s[g].wait()
            copies_g[g + NBUF] = gather(g + NBUF)
    for g in range(NCHUNK - NBUF, NCHUNK):
        copies_s[g].wait()


def kernel(x, table):
    ids = x.reshape(NW, NCHUNK, CHUNK).astype(jnp.int32)
    out = _emb_lookup(ids, table)
    return out.reshape(x.shape[0], x.shape[1], HIDDEN)
